# baseline (speedup 1.0000x reference)
import functools

import jax
import jax.numpy as jnp
from jax import lax
from jax.experimental import pallas as pl
from jax.experimental.pallas import tpu as pltpu
from jax.experimental.pallas import tpu_sc as plsc

NUM_CORES = 2
NUM_SUBCORES = 16
NW = NUM_CORES * NUM_SUBCORES
CH = 512


def _sc_mesh():
    return plsc.VectorSubcoreMesh(core_axis_name="c", subcore_axis_name="s")


def _deg_body(npad, chunks, dst_hbm, out_hbm, didx, ones_v, zbuf, sem, deg_sp):
    rows_per_tile = npad // NUM_SUBCORES
    c = lax.axis_index("c")
    s = lax.axis_index("s")
    wid = s * NUM_CORES + c

    @pl.loop(0, CH)
    def _(i):
        ones_v[i] = jnp.ones((16,), jnp.float32)

    @pl.loop(0, rows_per_tile)
    def _(i):
        zbuf[i] = jnp.zeros((16,), jnp.float32)

    pltpu.sync_copy(zbuf, deg_sp.at[pl.ds(s * rows_per_tile, rows_per_tile)])
    pltpu.sync_copy(dst_hbm.at[pl.ds(wid * chunks, chunks)], didx)
    plsc.subcore_barrier()

    for j in range(chunks):
        pltpu.async_copy(ones_v, deg_sp.at[didx.at[j]], sem, add=True)
    for j in range(chunks):
        pltpu.make_async_copy(ones_v, deg_sp.at[didx.at[0]], sem).wait()

    plsc.subcore_barrier()
    pltpu.sync_copy(deg_sp.at[pl.ds(s * rows_per_tile, rows_per_tile)], zbuf)
    pltpu.sync_copy(zbuf, out_hbm.at[c, pl.ds(s * rows_per_tile, rows_per_tile)])


def _agg_body(npad, chunks, hs_hbm, src_hbm, dst_hbm, out_hbm,
              sidx, didx, rows0, rows1, zbuf,
              sem_g0, sem_g1, sem_s0, sem_s1, agg_sp):
    rows_per_tile = npad // NUM_SUBCORES
    c = lax.axis_index("c")
    s = lax.axis_index("s")
    wid = s * NUM_CORES + c

    @pl.loop(0, rows_per_tile)
    def _(i):
        zbuf[i, pl.ds(0, 16)] = jnp.zeros((16,), jnp.float32)
        zbuf[i, pl.ds(16, 16)] = jnp.zeros((16,), jnp.float32)

    pltpu.sync_copy(zbuf, agg_sp.at[pl.ds(s * rows_per_tile, rows_per_tile)])
    pltpu.sync_copy(src_hbm.at[pl.ds(wid * chunks, chunks)], sidx)
    pltpu.sync_copy(dst_hbm.at[pl.ds(wid * chunks, chunks)], didx)
    plsc.subcore_barrier()

    slots = ((rows0, sem_g0, sem_s0), (rows1, sem_g1, sem_s1))
    nbuf = len(slots)
    for b, (rows, sg, _ss) in enumerate(slots):
        if b < chunks:
            pltpu.async_copy(hs_hbm.at[sidx.at[b]], rows, sg)

    @pl.loop(0, chunks // nbuf)
    def _(jo):
        for b, (rows, sg, ss) in enumerate(slots):
            j = jo * nbuf + b
            pltpu.make_async_copy(hs_hbm.at[sidx.at[0]], rows, sg).wait()
            pltpu.async_copy(rows, agg_sp.at[didx.at[j]], ss, add=True)
            nxt = j + nbuf

            @pl.when(nxt < chunks)
            def _():
                pltpu.make_async_copy(rows, agg_sp.at[didx.at[0]], ss).wait()
                pltpu.async_copy(hs_hbm.at[sidx.at[nxt]], rows, sg)

    for b, (rows, _sg, ss) in enumerate(slots):
        pltpu.make_async_copy(rows, agg_sp.at[didx.at[0]], ss).wait()

    plsc.subcore_barrier()
    pltpu.sync_copy(agg_sp.at[pl.ds(s * rows_per_tile, rows_per_tile)], zbuf)
    pltpu.sync_copy(zbuf, out_hbm.at[c, pl.ds(s * rows_per_tile, rows_per_tile)])


def _sc_degree(dst2d, npad, chunks):
    body = functools.partial(_deg_body, npad, chunks)
    rows_per_tile = npad // NUM_SUBCORES
    return pl.kernel(
        body,
        out_type=jax.ShapeDtypeStruct((NUM_CORES, npad, 16), jnp.float32),
        mesh=_sc_mesh(),
        scratch_types=[
            pltpu.VMEM((chunks, CH), jnp.int32),
            pltpu.VMEM((CH, 16), jnp.float32),
            pltpu.VMEM((rows_per_tile, 16), jnp.float32),
            pltpu.SemaphoreType.DMA,
            pltpu.VMEM_SHARED((npad, 16), jnp.float32),
        ],
        compiler_params=pltpu.CompilerParams(use_tc_tiling_on_sc=False),
    )(dst2d)


def _sc_aggregate(hs, src2d, dst2d, npad, chunks):
    body = functools.partial(_agg_body, npad, chunks)
    rows_per_tile = npad // NUM_SUBCORES
    return pl.kernel(
        body,
        out_type=jax.ShapeDtypeStruct((NUM_CORES, npad, 32), jnp.float32),
        mesh=_sc_mesh(),
        scratch_types=[
            pltpu.VMEM((chunks, CH), jnp.int32),
            pltpu.VMEM((chunks, CH), jnp.int32),
            pltpu.VMEM((CH, 32), jnp.float32),
            pltpu.VMEM((CH, 32), jnp.float32),
            pltpu.VMEM((rows_per_tile, 32), jnp.float32),
            pltpu.SemaphoreType.DMA,
            pltpu.SemaphoreType.DMA,
            pltpu.SemaphoreType.DMA,
            pltpu.SemaphoreType.DMA,
            pltpu.VMEM_SHARED((npad, 32), jnp.float32),
        ],
        compiler_params=pltpu.CompilerParams(use_tc_tiling_on_sc=False),
    )(hs, src2d, dst2d)


def _mm_scale_body(x_ref, w_ref, degp_ref, dinv_ref, hs1_ref):
    h1 = jnp.dot(x_ref[:], w_ref[:], preferred_element_type=jnp.float32)
    deg = degp_ref[0] + degp_ref[1] + 1.0
    dinv = lax.rsqrt(deg)
    dinv_ref[:] = dinv
    hs1_ref[:] = h1 * dinv[:, 0:1]


def _layer1_body(p0_ref, p1_ref, hs1_ref, dinv_ref, b1_ref, hs2_ref):
    d = dinv_ref[:, 0:1]
    agg = d * (p0_ref[0] + p1_ref[0] + hs1_ref[:])
    h = jnp.maximum(agg + b1_ref[:], 0.0)
    hs2_ref[:] = d * h


def _latent_block(p2, hs2, dinv, eps, wmu, bmu, wlv, blv):
    d = dinv[:, 0:1]
    agg2 = d * (p2[0] + p2[1] + hs2)
    mu = jnp.dot(agg2, wmu, preferred_element_type=jnp.float32) + bmu
    lv = jnp.dot(agg2, wlv, preferred_element_type=jnp.float32) + blv
    z = mu + jnp.exp(lv) * eps
    return mu, lv, z


def _dec_body(p2r_ref, p2c_ref, hs2r_ref, hs2c_ref, dinvr_ref, dinvc_ref,
              epsr_ref, epsc_ref, wmu_ref, bmu_ref, wlv_ref, blv_ref,
              adj_ref, mu_ref, lv_ref):
    wmu, bmu, wlv, blv = wmu_ref[:], bmu_ref[:], wlv_ref[:], blv_ref[:]
    mu, lv, z_r = _latent_block(p2r_ref[:], hs2r_ref[:], dinvr_ref[:],
                                epsr_ref[:], wmu, bmu, wlv, blv)
    _, _, z_c = _latent_block(p2c_ref[:], hs2c_ref[:], dinvc_ref[:],
                              epsc_ref[:], wmu, bmu, wlv, blv)
    mu_ref[:] = mu
    lv_ref[:] = lv
    logits = lax.dot_general(z_r, z_c, (((1,), (1,)), ((), ())),
                             preferred_element_type=jnp.float32)
    adj_ref[:] = 0.5 * jnp.tanh(0.5 * logits) + 0.5


def kernel(x, edge_index, W1, b1, Wmu, bmu, Wlv, blv, eps):
    n = x.shape[0]
    nfeat = x.shape[1]
    nhid = W1.shape[1]
    latent = Wmu.shape[1]
    e = edge_index.shape[1]

    npad = ((n + NUM_SUBCORES * 16 - 1) // (NUM_SUBCORES * 16)) * NUM_SUBCORES * 16
    if npad == n:
        npad = n + NUM_SUBCORES * 16
    epad = ((e + NW * CH - 1) // (NW * CH)) * NW * CH
    chunks = epad // (NW * CH)

    src = edge_index[0]
    dst = edge_index[1]
    pad = epad - e
    pad_src = jnp.arange(pad, dtype=src.dtype) % n
    src2d = jnp.concatenate([src, pad_src]).reshape(-1, CH)
    pad_dst = n + (jnp.arange(pad, dtype=dst.dtype) % (npad - n))
    dst2d = jnp.concatenate([dst, pad_dst]).reshape(-1, CH)

    bm = 2000
    grid_r = n // bm

    deg_part = _sc_degree(dst2d, npad, chunks)

    dinv, hs1 = pl.pallas_call(
        _mm_scale_body,
        grid=(grid_r,),
        in_specs=[pl.BlockSpec((bm, nfeat), lambda i: (i, 0)),
                  pl.BlockSpec((nfeat, nhid), lambda i: (0, 0)),
                  pl.BlockSpec((2, bm, 16), lambda i: (0, i, 0))],
        out_specs=[pl.BlockSpec((bm, 16), lambda i: (i, 0)),
                   pl.BlockSpec((bm, nhid), lambda i: (i, 0))],
        out_shape=[jax.ShapeDtypeStruct((n, 16), jnp.float32),
                   jax.ShapeDtypeStruct((n, nhid), jnp.float32)],
    )(x, W1, deg_part)

    p1 = _sc_aggregate(hs1, src2d, dst2d, npad, chunks)

    hs2 = pl.pallas_call(
        _layer1_body,
        grid=(grid_r,),
        in_specs=[pl.BlockSpec((1, bm, nhid), lambda i: (0, i, 0)),
                  pl.BlockSpec((1, bm, nhid), lambda i: (1, i, 0)),
                  pl.BlockSpec((bm, nhid), lambda i: (i, 0)),
                  pl.BlockSpec((bm, 16), lambda i: (i, 0)),
                  pl.BlockSpec((1, nhid), lambda i: (0, 0))],
        out_specs=pl.BlockSpec((bm, nhid), lambda i: (i, 0)),
        out_shape=jax.ShapeDtypeStruct((n, nhid), jnp.float32),
    )(p1, p1, hs1, dinv, b1.reshape(1, nhid))

    p2 = _sc_aggregate(hs2, src2d, dst2d, npad, chunks)

    bmr, bnc = 2048, 2048
    gr = (n + bmr - 1) // bmr
    gc = (n + bnc - 1) // bnc
    adj, mu, logvar = pl.pallas_call(
        _dec_body,
        grid=(gr, gc),
        in_specs=[pl.BlockSpec((2, bmr, nhid), lambda i, j: (0, i, 0)),
                  pl.BlockSpec((2, bnc, nhid), lambda i, j: (0, j, 0)),
                  pl.BlockSpec((bmr, nhid), lambda i, j: (i, 0)),
                  pl.BlockSpec((bnc, nhid), lambda i, j: (j, 0)),
                  pl.BlockSpec((bmr, 16), lambda i, j: (i, 0)),
                  pl.BlockSpec((bnc, 16), lambda i, j: (j, 0)),
                  pl.BlockSpec((bmr, latent), lambda i, j: (i, 0)),
                  pl.BlockSpec((bnc, latent), lambda i, j: (j, 0)),
                  pl.BlockSpec((nhid, latent), lambda i, j: (0, 0)),
                  pl.BlockSpec((1, latent), lambda i, j: (0, 0)),
                  pl.BlockSpec((nhid, latent), lambda i, j: (0, 0)),
                  pl.BlockSpec((1, latent), lambda i, j: (0, 0))],
        out_specs=[pl.BlockSpec((bmr, bnc), lambda i, j: (i, j)),
                   pl.BlockSpec((bmr, latent), lambda i, j: (i, 0)),
                   pl.BlockSpec((bmr, latent), lambda i, j: (i, 0))],
        out_shape=[jax.ShapeDtypeStruct((n, n), jnp.float32),
                   jax.ShapeDtypeStruct((n, latent), jnp.float32),
                   jax.ShapeDtypeStruct((n, latent), jnp.float32)],
    )(p2, p2, hs2, hs2, dinv, dinv, eps, eps,
      Wmu, bmu.reshape(1, latent), Wlv, blv.reshape(1, latent))

    return (adj, mu, logvar)


# speedup vs baseline: 1.0978x; 1.0978x over previous
import functools

import jax
import jax.numpy as jnp
from jax import lax
from jax.experimental import pallas as pl
from jax.experimental.pallas import tpu as pltpu
from jax.experimental.pallas import tpu_sc as plsc

NUM_CORES = 2
NUM_SUBCORES = 16
NW = NUM_CORES * NUM_SUBCORES
CH = 512


def _sc_mesh():
    return plsc.VectorSubcoreMesh(core_axis_name="c", subcore_axis_name="s")


def _deg_body(npad, chunks, dst_hbm, out_hbm, didx, ones_v, zbuf, sem, deg_sp):
    rows_per_tile = npad // NUM_SUBCORES
    c = lax.axis_index("c")
    s = lax.axis_index("s")
    wid = s * NUM_CORES + c

    @pl.loop(0, CH)
    def _(i):
        ones_v[i] = jnp.ones((16,), jnp.float32)

    @pl.loop(0, rows_per_tile)
    def _(i):
        zbuf[i] = jnp.zeros((16,), jnp.float32)

    pltpu.sync_copy(zbuf, deg_sp.at[pl.ds(s * rows_per_tile, rows_per_tile)])
    pltpu.sync_copy(dst_hbm.at[pl.ds(wid * chunks, chunks)], didx)
    plsc.subcore_barrier()

    for j in range(chunks):
        pltpu.async_copy(ones_v, deg_sp.at[didx.at[j]], sem, add=True)
    for j in range(chunks):
        pltpu.make_async_copy(ones_v, deg_sp.at[didx.at[0]], sem).wait()

    plsc.subcore_barrier()
    pltpu.sync_copy(deg_sp.at[pl.ds(s * rows_per_tile, rows_per_tile)], zbuf)
    pltpu.sync_copy(zbuf, out_hbm.at[c, pl.ds(s * rows_per_tile, rows_per_tile)])


def _agg_body(npad, chunks, hs_hbm, src_hbm, dst_hbm, out_hbm,
              sidx, didx, rows0, rows1, zbuf,
              sem_g0, sem_g1, sem_s0, sem_s1, agg_sp):
    rows_per_tile = npad // NUM_SUBCORES
    c = lax.axis_index("c")
    s = lax.axis_index("s")
    wid = s * NUM_CORES + c

    @pl.loop(0, rows_per_tile)
    def _(i):
        zbuf[i, pl.ds(0, 16)] = jnp.zeros((16,), jnp.float32)
        zbuf[i, pl.ds(16, 16)] = jnp.zeros((16,), jnp.float32)

    pltpu.sync_copy(zbuf, agg_sp.at[pl.ds(s * rows_per_tile, rows_per_tile)])
    pltpu.sync_copy(src_hbm.at[pl.ds(wid * chunks, chunks)], sidx)
    pltpu.sync_copy(dst_hbm.at[pl.ds(wid * chunks, chunks)], didx)
    plsc.subcore_barrier()

    slots = ((rows0, sem_g0, sem_s0), (rows1, sem_g1, sem_s1))
    nbuf = len(slots)
    for b, (rows, sg, _ss) in enumerate(slots):
        if b < chunks:
            pltpu.async_copy(hs_hbm.at[sidx.at[b]], rows, sg)

    @pl.loop(0, chunks // nbuf)
    def _(jo):
        for b, (rows, sg, ss) in enumerate(slots):
            j = jo * nbuf + b
            pltpu.make_async_copy(hs_hbm.at[sidx.at[0]], rows, sg).wait()
            pltpu.async_copy(rows, agg_sp.at[didx.at[j]], ss, add=True)
            nxt = j + nbuf

            @pl.when(nxt < chunks)
            def _():
                pltpu.make_async_copy(rows, agg_sp.at[didx.at[0]], ss).wait()
                pltpu.async_copy(hs_hbm.at[sidx.at[nxt]], rows, sg)

    for b, (rows, _sg, ss) in enumerate(slots):
        pltpu.make_async_copy(rows, agg_sp.at[didx.at[0]], ss).wait()

    plsc.subcore_barrier()
    pltpu.sync_copy(agg_sp.at[pl.ds(s * rows_per_tile, rows_per_tile)], zbuf)
    pltpu.sync_copy(zbuf, out_hbm.at[c, pl.ds(s * rows_per_tile, rows_per_tile)])


def _sc_degree(dst2d, npad, chunks):
    body = functools.partial(_deg_body, npad, chunks)
    rows_per_tile = npad // NUM_SUBCORES
    return pl.kernel(
        body,
        out_type=jax.ShapeDtypeStruct((NUM_CORES, npad, 16), jnp.float32),
        mesh=_sc_mesh(),
        scratch_types=[
            pltpu.VMEM((chunks, CH), jnp.int32),
            pltpu.VMEM((CH, 16), jnp.float32),
            pltpu.VMEM((rows_per_tile, 16), jnp.float32),
            pltpu.SemaphoreType.DMA,
            pltpu.VMEM_SHARED((npad, 16), jnp.float32),
        ],
        compiler_params=pltpu.CompilerParams(use_tc_tiling_on_sc=False),
    )(dst2d)


def _sc_aggregate(hs, src2d, dst2d, npad, chunks):
    body = functools.partial(_agg_body, npad, chunks)
    rows_per_tile = npad // NUM_SUBCORES
    return pl.kernel(
        body,
        out_type=jax.ShapeDtypeStruct((NUM_CORES, npad, 32), jnp.float32),
        mesh=_sc_mesh(),
        scratch_types=[
            pltpu.VMEM((chunks, CH), jnp.int32),
            pltpu.VMEM((chunks, CH), jnp.int32),
            pltpu.VMEM((CH, 32), jnp.float32),
            pltpu.VMEM((CH, 32), jnp.float32),
            pltpu.VMEM((rows_per_tile, 32), jnp.float32),
            pltpu.SemaphoreType.DMA,
            pltpu.SemaphoreType.DMA,
            pltpu.SemaphoreType.DMA,
            pltpu.SemaphoreType.DMA,
            pltpu.VMEM_SHARED((npad, 32), jnp.float32),
        ],
        compiler_params=pltpu.CompilerParams(use_tc_tiling_on_sc=False),
    )(hs, src2d, dst2d)


def _mm_scale_body(x_ref, w_ref, degp_ref, dinv_ref, hs1_ref):
    h1 = jnp.dot(x_ref[:], w_ref[:], preferred_element_type=jnp.float32)
    deg = degp_ref[0] + degp_ref[1] + 1.0
    dinv = lax.rsqrt(deg)
    dinv_ref[:] = dinv
    hs1_ref[:] = h1 * dinv[:, 0:1]


def _layer1_body(p0_ref, p1_ref, hs1_ref, dinv_ref, b1_ref, hs2_ref):
    d = dinv_ref[:, 0:1]
    agg = d * (p0_ref[0] + p1_ref[0] + hs1_ref[:])
    h = jnp.maximum(agg + b1_ref[:], 0.0)
    hs2_ref[:] = d * h


def _latent_body(p0_ref, p1_ref, hs2_ref, dinv_ref, wmu_ref, bmu_ref,
                 wlv_ref, blv_ref, eps_ref, mu_ref, lv_ref, z_ref):
    d = dinv_ref[:, 0:1]
    agg2 = d * (p0_ref[0] + p1_ref[0] + hs2_ref[:])
    mu = jnp.dot(agg2, wmu_ref[:], preferred_element_type=jnp.float32) + bmu_ref[:]
    lv = jnp.dot(agg2, wlv_ref[:], preferred_element_type=jnp.float32) + blv_ref[:]
    mu_ref[:] = mu
    lv_ref[:] = lv
    z_ref[:] = mu + jnp.exp(lv) * eps_ref[:]


def _dec_body(zr_ref, zc_ref, o_ref):
    logits = lax.dot_general(zr_ref[:], zc_ref[:], (((1,), (1,)), ((), ())),
                             preferred_element_type=jnp.float32)
    o_ref[:] = 0.5 * jnp.tanh(0.5 * logits) + 0.5


def kernel(x, edge_index, W1, b1, Wmu, bmu, Wlv, blv, eps):
    n = x.shape[0]
    nfeat = x.shape[1]
    nhid = W1.shape[1]
    latent = Wmu.shape[1]
    e = edge_index.shape[1]

    npad = ((n + NUM_SUBCORES * 16 - 1) // (NUM_SUBCORES * 16)) * NUM_SUBCORES * 16
    if npad == n:
        npad = n + NUM_SUBCORES * 16
    epad = ((e + NW * CH - 1) // (NW * CH)) * NW * CH
    chunks = epad // (NW * CH)

    src = edge_index[0]
    dst = edge_index[1]
    pad = epad - e
    pad_src = jnp.arange(pad, dtype=src.dtype) % n
    src2d = jnp.concatenate([src, pad_src]).reshape(-1, CH)
    pad_dst = n + (jnp.arange(pad, dtype=dst.dtype) % (npad - n))
    dst2d = jnp.concatenate([dst, pad_dst]).reshape(-1, CH)

    bm = 2000
    grid_r = n // bm

    deg_part = _sc_degree(dst2d, npad, chunks)

    dinv, hs1 = pl.pallas_call(
        _mm_scale_body,
        grid=(grid_r,),
        in_specs=[pl.BlockSpec((bm, nfeat), lambda i: (i, 0)),
                  pl.BlockSpec((nfeat, nhid), lambda i: (0, 0)),
                  pl.BlockSpec((2, bm, 16), lambda i: (0, i, 0))],
        out_specs=[pl.BlockSpec((bm, 16), lambda i: (i, 0)),
                   pl.BlockSpec((bm, nhid), lambda i: (i, 0))],
        out_shape=[jax.ShapeDtypeStruct((n, 16), jnp.float32),
                   jax.ShapeDtypeStruct((n, nhid), jnp.float32)],
    )(x, W1, deg_part)

    p1 = _sc_aggregate(hs1, src2d, dst2d, npad, chunks)

    hs2 = pl.pallas_call(
        _layer1_body,
        grid=(grid_r,),
        in_specs=[pl.BlockSpec((1, bm, nhid), lambda i: (0, i, 0)),
                  pl.BlockSpec((1, bm, nhid), lambda i: (1, i, 0)),
                  pl.BlockSpec((bm, nhid), lambda i: (i, 0)),
                  pl.BlockSpec((bm, 16), lambda i: (i, 0)),
                  pl.BlockSpec((1, nhid), lambda i: (0, 0))],
        out_specs=pl.BlockSpec((bm, nhid), lambda i: (i, 0)),
        out_shape=jax.ShapeDtypeStruct((n, nhid), jnp.float32),
    )(p1, p1, hs1, dinv, b1.reshape(1, nhid))

    p2 = _sc_aggregate(hs2, src2d, dst2d, npad, chunks)

    mu, logvar, z = pl.pallas_call(
        _latent_body,
        grid=(grid_r,),
        in_specs=[pl.BlockSpec((1, bm, nhid), lambda i: (0, i, 0)),
                  pl.BlockSpec((1, bm, nhid), lambda i: (1, i, 0)),
                  pl.BlockSpec((bm, nhid), lambda i: (i, 0)),
                  pl.BlockSpec((bm, 16), lambda i: (i, 0)),
                  pl.BlockSpec((nhid, latent), lambda i: (0, 0)),
                  pl.BlockSpec((1, latent), lambda i: (0, 0)),
                  pl.BlockSpec((nhid, latent), lambda i: (0, 0)),
                  pl.BlockSpec((1, latent), lambda i: (0, 0)),
                  pl.BlockSpec((bm, latent), lambda i: (i, 0))],
        out_specs=[pl.BlockSpec((bm, latent), lambda i: (i, 0)),
                   pl.BlockSpec((bm, latent), lambda i: (i, 0)),
                   pl.BlockSpec((bm, latent), lambda i: (i, 0))],
        out_shape=[jax.ShapeDtypeStruct((n, latent), jnp.float32),
                   jax.ShapeDtypeStruct((n, latent), jnp.float32),
                   jax.ShapeDtypeStruct((n, latent), jnp.float32)],
    )(p2, p2, hs2, dinv, Wmu, bmu.reshape(1, latent), Wlv, blv.reshape(1, latent), eps)

    bmr, bnc = 2048, 2048
    gr = (n + bmr - 1) // bmr
    gc = (n + bnc - 1) // bnc
    adj = pl.pallas_call(
        _dec_body,
        grid=(gr, gc),
        in_specs=[pl.BlockSpec((bmr, latent), lambda i, j: (i, 0)),
                  pl.BlockSpec((bnc, latent), lambda i, j: (j, 0))],
        out_specs=pl.BlockSpec((bmr, bnc), lambda i, j: (i, j)),
        out_shape=jax.ShapeDtypeStruct((n, n), jnp.float32),
    )(z, z)

    return (adj, mu, logvar)
